# baseline (device time: 102831 ns/iter reference)
import jax
import jax.numpy as jnp
from jax import lax
from jax.experimental import pallas as pl
from jax.experimental.pallas import tpu as pltpu

M = 8192
N_OUT = 1024
QROWS = M // 4
CH = 128
KQ = QROWS // CH
NCHUNKS = M // CH

DIAG_VIA_X = (0, 1, 2, 3, 4)
DIAG_VIA_Z = (5, 6, 7, 8, 9)
DIAG_VIA_Y = (10, 11, 12, 13, 14, 15)
KY = KQ + len(DIAG_VIA_Y)
KX = KQ + len(DIAG_VIA_X)
KZ = KQ + len(DIAG_VIA_Z)


def kernel(x):

    def body(
        x_ref,
        out_ref,
        rbuf,
        ysend,
        lstage_s,
        lstage_k,
        obuf,
        csem_s,
        csem_k,
        osem,
        ssem_y,
        rsem_y,
        ssem_x,
        rsem_x,
        ssem_z,
        rsem_z,
    ):
        my_x = lax.axis_index("x")
        my_y = lax.axis_index("y")
        my_z = lax.axis_index("z")
        yn = (my_x, 1 - my_y, my_z)
        xn = (1 - my_x, my_y, my_z)
        zn = (my_x, my_y, 1 - my_z)
        qme = 2 * my_z + my_x
        qxn = 2 * my_z + (1 - my_x)
        qzn = 2 * (1 - my_z) + my_x
        qdg = 2 * (1 - my_z) + (1 - my_x)
        send_col = (1 - my_y) * N_OUT
        keep_col = my_y * N_OUT

        def start_stage(q, c, col, stage, sem, slot):
            cp = pltpu.make_async_copy(
                x_ref.at[0, pl.ds((q * KQ + c) * CH, CH), pl.ds(col, N_OUT)],
                stage.at[slot],
                sem.at[slot],
            )
            cp.start()
            return cp

        def swap_rdma(j, ssem, rsem, si, target):
            return pltpu.make_async_remote_copy(
                src_ref=rbuf.at[j],
                dst_ref=rbuf.at[j],
                send_sem=ssem.at[si],
                recv_sem=rsem.at[si],
                device_id=target,
                device_id_type=pl.DeviceIdType.MESH,
            )

        oc = [0]
        pend = [None] * 4

        def add_chunk(j, slot):
            s = oc[0] % 4
            if pend[s] is not None:
                pend[s].wait()
            obuf[s, :, :] = lstage_k[slot].astype(jnp.bfloat16) + rbuf[j]
            cp = pltpu.make_async_copy(
                obuf.at[s], out_ref.at[pl.ds(j * CH, CH), :], osem.at[s]
            )
            cp.start()
            pend[s] = cp
            oc[0] += 1

        def run_phase(items):
            n = len(items)
            ks = [None] * n
            q0, c0, _ = items[0]
            ks[0] = start_stage(q0, c0, keep_col, lstage_k, csem_k, 0)
            for i, (q, c, handler) in enumerate(items):
                if i + 1 < n:
                    qn_, cn_, _ = items[i + 1]
                    ks[i + 1] = start_stage(
                        qn_, cn_, keep_col, lstage_k, csem_k, (i + 1) % 2
                    )
                handler()
                ks[i].wait()
                add_chunk(q * KQ + c, i % 2)

        aseq = [(qme, c) for c in range(KQ)] + [(qdg, c) for c in DIAG_VIA_Y]
        rdy = [None] * KY
        cps = [None] * KY
        cps[0] = start_stage(aseq[0][0], aseq[0][1], send_col, lstage_s, csem_s, 0)
        for i, (q, c) in enumerate(aseq):
            if i + 1 < KY:
                qn_, cn_ = aseq[i + 1]
                cps[i + 1] = start_stage(
                    qn_, cn_, send_col, lstage_s, csem_s, (i + 1) % 2
                )
            cps[i].wait()
            ysend[i, :, :] = lstage_s[i % 2].astype(jnp.bfloat16)
            rd = pltpu.make_async_remote_copy(
                src_ref=ysend.at[i],
                dst_ref=rbuf.at[q * KQ + c],
                send_sem=ssem_y.at[i],
                recv_sem=rsem_y.at[i],
                device_id=yn,
                device_id_type=pl.DeviceIdType.MESH,
            )
            rd.start()
            rdy[i] = rd

        rdx_out = [None] * KX
        rdz_out = [None] * KZ

        def handle_b(c):
            def h():
                rdy[c].wait_recv()
                j = qme * KQ + c
                rdx_out[c] = swap_rdma(j, ssem_x, rsem_x, c, xn)
                rdx_out[c].start()
                rdz_out[c] = swap_rdma(j, ssem_z, rsem_z, c, zn)
                rdz_out[c].start()
            return h

        run_phase([(qme, c, handle_b(c)) for c in range(KQ)])

        def handle_cx(c):
            def h():
                j = qxn * KQ + c
                swap_rdma(j, ssem_x, rsem_x, c, xn).wait_recv()
                if c in DIAG_VIA_Z:
                    si = KQ + DIAG_VIA_Z.index(c)
                    rdz_out[si] = swap_rdma(j, ssem_z, rsem_z, si, zn)
                    rdz_out[si].start()
            return h

        def handle_cz(c):
            def h():
                j = qzn * KQ + c
                swap_rdma(j, ssem_z, rsem_z, c, zn).wait_recv()
                if c in DIAG_VIA_X:
                    si = KQ + DIAG_VIA_X.index(c)
                    rdx_out[si] = swap_rdma(j, ssem_x, rsem_x, si, xn)
                    rdx_out[si].start()
            return h

        citems = []
        for c in range(KQ):
            citems.append((qxn, c, handle_cx(c)))
            citems.append((qzn, c, handle_cz(c)))
        run_phase(citems)

        def handle_d(c):
            def h():
                j = qdg * KQ + c
                if c in DIAG_VIA_Y:
                    rdy[KQ + DIAG_VIA_Y.index(c)].wait_recv()
                elif c in DIAG_VIA_Z:
                    swap_rdma(
                        j, ssem_z, rsem_z, KQ + DIAG_VIA_Z.index(c), zn
                    ).wait_recv()
                else:
                    swap_rdma(
                        j, ssem_x, rsem_x, KQ + DIAG_VIA_X.index(c), xn
                    ).wait_recv()
            return h

        dorder = list(DIAG_VIA_Y) + list(DIAG_VIA_Z) + list(DIAG_VIA_X)
        run_phase([(qdg, c, handle_d(c)) for c in dorder])

        for rd in rdy:
            rd.wait_send()
        for rd in rdx_out:
            rd.wait_send()
        for rd in rdz_out:
            rd.wait_send()
        for cp in pend:
            cp.wait()

    return pl.pallas_call(
        body,
        out_shape=jax.ShapeDtypeStruct((M, N_OUT), jnp.bfloat16),
        in_specs=[pl.BlockSpec(memory_space=pl.ANY)],
        out_specs=pl.BlockSpec(memory_space=pl.ANY),
        scratch_shapes=[
            pltpu.VMEM((NCHUNKS, CH, N_OUT), jnp.bfloat16),
            pltpu.VMEM((KY, CH, N_OUT), jnp.bfloat16),
            pltpu.VMEM((2, CH, N_OUT), jnp.float32),
            pltpu.VMEM((2, CH, N_OUT), jnp.float32),
            pltpu.VMEM((4, CH, N_OUT), jnp.bfloat16),
            pltpu.SemaphoreType.DMA((2,)),
            pltpu.SemaphoreType.DMA((2,)),
            pltpu.SemaphoreType.DMA((4,)),
            pltpu.SemaphoreType.DMA((KY,)),
            pltpu.SemaphoreType.DMA((KY,)),
            pltpu.SemaphoreType.DMA((KX,)),
            pltpu.SemaphoreType.DMA((KX,)),
            pltpu.SemaphoreType.DMA((KZ,)),
            pltpu.SemaphoreType.DMA((KZ,)),
        ],
        compiler_params=pltpu.CompilerParams(
            vmem_limit_bytes=60 * 1024 * 1024,
        ),
    )(x)


# device time: 91086 ns/iter; 1.1289x vs baseline; 1.1289x over previous
import jax
import jax.numpy as jnp
from jax import lax
from jax.experimental import pallas as pl
from jax.experimental.pallas import tpu as pltpu

M = 8192
N_OUT = 1024
QROWS = M // 4
CH = 256
KQ = QROWS // CH
NCHUNKS = M // CH

DIAG_VIA_X = (0, 1, 2)
DIAG_VIA_Z = (3, 4)
DIAG_VIA_Y = (5, 6, 7)
KY = KQ + len(DIAG_VIA_Y)
KX = KQ + len(DIAG_VIA_X)
KZ = KQ + len(DIAG_VIA_Z)


def kernel(x):

    def body(
        x_ref,
        out_ref,
        rbuf,
        ysend,
        lstage_s,
        lstage_k,
        obuf,
        csem_s,
        csem_k,
        osem,
        ssem_y,
        rsem_y,
        ssem_x,
        rsem_x,
        ssem_z,
        rsem_z,
    ):
        my_x = lax.axis_index("x")
        my_y = lax.axis_index("y")
        my_z = lax.axis_index("z")
        yn = (my_x, 1 - my_y, my_z)
        xn = (1 - my_x, my_y, my_z)
        zn = (my_x, my_y, 1 - my_z)
        qme = 2 * my_z + my_x
        qxn = 2 * my_z + (1 - my_x)
        qzn = 2 * (1 - my_z) + my_x
        qdg = 2 * (1 - my_z) + (1 - my_x)
        send_col = (1 - my_y) * N_OUT
        keep_col = my_y * N_OUT

        barrier_sem = pltpu.get_barrier_semaphore()
        for nbr in (yn, xn, zn):
            pl.semaphore_signal(
                barrier_sem,
                inc=1,
                device_id=nbr,
                device_id_type=pl.DeviceIdType.MESH,
            )
        pl.semaphore_wait(barrier_sem, 3)

        def start_stage(q, c, col, stage, sem, slot):
            cp = pltpu.make_async_copy(
                x_ref.at[0, pl.ds((q * KQ + c) * CH, CH), pl.ds(col, N_OUT)],
                stage.at[slot],
                sem.at[slot],
            )
            cp.start()
            return cp

        def swap_rdma(j, ssem, rsem, si, target):
            return pltpu.make_async_remote_copy(
                src_ref=rbuf.at[j],
                dst_ref=rbuf.at[j],
                send_sem=ssem.at[si],
                recv_sem=rsem.at[si],
                device_id=target,
                device_id_type=pl.DeviceIdType.MESH,
            )

        oc = [0]
        pend = [None] * 4

        def add_chunk(j, slot):
            s = oc[0] % 4
            if pend[s] is not None:
                pend[s].wait()
            obuf[s, :, :] = lstage_k[slot].astype(jnp.bfloat16) + rbuf[j]
            cp = pltpu.make_async_copy(
                obuf.at[s], out_ref.at[pl.ds(j * CH, CH), :], osem.at[s]
            )
            cp.start()
            pend[s] = cp
            oc[0] += 1

        def run_phase(items):
            n = len(items)
            ks = [None] * n
            q0, c0, _ = items[0]
            ks[0] = start_stage(q0, c0, keep_col, lstage_k, csem_k, 0)
            for i, (q, c, handler) in enumerate(items):
                if i + 1 < n:
                    qn_, cn_, _ = items[i + 1]
                    ks[i + 1] = start_stage(
                        qn_, cn_, keep_col, lstage_k, csem_k, (i + 1) % 2
                    )
                handler()
                ks[i].wait()
                add_chunk(q * KQ + c, i % 2)

        aseq = [(qme, c) for c in range(KQ)] + [(qdg, c) for c in DIAG_VIA_Y]
        rdy = [None] * KY
        cps = [None] * KY
        cps[0] = start_stage(aseq[0][0], aseq[0][1], send_col, lstage_s, csem_s, 0)
        for i, (q, c) in enumerate(aseq):
            if i + 1 < KY:
                qn_, cn_ = aseq[i + 1]
                cps[i + 1] = start_stage(
                    qn_, cn_, send_col, lstage_s, csem_s, (i + 1) % 2
                )
            cps[i].wait()
            ysend[i, :, :] = lstage_s[i % 2].astype(jnp.bfloat16)
            rd = pltpu.make_async_remote_copy(
                src_ref=ysend.at[i],
                dst_ref=rbuf.at[q * KQ + c],
                send_sem=ssem_y.at[i],
                recv_sem=rsem_y.at[i],
                device_id=yn,
                device_id_type=pl.DeviceIdType.MESH,
            )
            rd.start()
            rdy[i] = rd

        rdx_out = [None] * KX
        rdz_out = [None] * KZ

        def handle_b(c):
            def h():
                rdy[c].wait_recv()
                j = qme * KQ + c
                rdx_out[c] = swap_rdma(j, ssem_x, rsem_x, c, xn)
                rdx_out[c].start()
                rdz_out[c] = swap_rdma(j, ssem_z, rsem_z, c, zn)
                rdz_out[c].start()
            return h

        run_phase([(qme, c, handle_b(c)) for c in range(KQ)])

        def handle_cx(c):
            def h():
                j = qxn * KQ + c
                swap_rdma(j, ssem_x, rsem_x, c, xn).wait_recv()
                if c in DIAG_VIA_Z:
                    si = KQ + DIAG_VIA_Z.index(c)
                    rdz_out[si] = swap_rdma(j, ssem_z, rsem_z, si, zn)
                    rdz_out[si].start()
            return h

        def handle_cz(c):
            def h():
                j = qzn * KQ + c
                swap_rdma(j, ssem_z, rsem_z, c, zn).wait_recv()
                if c in DIAG_VIA_X:
                    si = KQ + DIAG_VIA_X.index(c)
                    rdx_out[si] = swap_rdma(j, ssem_x, rsem_x, si, xn)
                    rdx_out[si].start()
            return h

        citems = []
        for c in range(KQ):
            citems.append((qxn, c, handle_cx(c)))
            citems.append((qzn, c, handle_cz(c)))
        run_phase(citems)

        def handle_d(c):
            def h():
                j = qdg * KQ + c
                if c in DIAG_VIA_Y:
                    rdy[KQ + DIAG_VIA_Y.index(c)].wait_recv()
                elif c in DIAG_VIA_Z:
                    swap_rdma(
                        j, ssem_z, rsem_z, KQ + DIAG_VIA_Z.index(c), zn
                    ).wait_recv()
                else:
                    swap_rdma(
                        j, ssem_x, rsem_x, KQ + DIAG_VIA_X.index(c), xn
                    ).wait_recv()
            return h

        dorder = list(DIAG_VIA_Y) + list(DIAG_VIA_Z) + list(DIAG_VIA_X)
        run_phase([(qdg, c, handle_d(c)) for c in dorder])

        for rd in rdy:
            rd.wait_send()
        for rd in rdx_out:
            rd.wait_send()
        for rd in rdz_out:
            rd.wait_send()
        for cp in pend:
            cp.wait()

    return pl.pallas_call(
        body,
        out_shape=jax.ShapeDtypeStruct((M, N_OUT), jnp.bfloat16),
        in_specs=[pl.BlockSpec(memory_space=pl.ANY)],
        out_specs=pl.BlockSpec(memory_space=pl.ANY),
        scratch_shapes=[
            pltpu.VMEM((NCHUNKS, CH, N_OUT), jnp.bfloat16),
            pltpu.VMEM((KY, CH, N_OUT), jnp.bfloat16),
            pltpu.VMEM((2, CH, N_OUT), jnp.float32),
            pltpu.VMEM((2, CH, N_OUT), jnp.float32),
            pltpu.VMEM((4, CH, N_OUT), jnp.bfloat16),
            pltpu.SemaphoreType.DMA((2,)),
            pltpu.SemaphoreType.DMA((2,)),
            pltpu.SemaphoreType.DMA((4,)),
            pltpu.SemaphoreType.DMA((KY,)),
            pltpu.SemaphoreType.DMA((KY,)),
            pltpu.SemaphoreType.DMA((KX,)),
            pltpu.SemaphoreType.DMA((KX,)),
            pltpu.SemaphoreType.DMA((KZ,)),
            pltpu.SemaphoreType.DMA((KZ,)),
        ],
        compiler_params=pltpu.CompilerParams(
            vmem_limit_bytes=60 * 1024 * 1024,
            collective_id=0,
        ),
    )(x)
